# Initial kernel scaffold; baseline (speedup 1.0000x reference)
import jax
import jax.numpy as jnp
from jax.experimental import pallas as pl


def kernel(time, event, time_table, event_table, pos_table):
    raise NotImplementedError("write your pallas kernel here")


# speedup vs baseline: 3.9429x; 3.9429x over previous
import functools

import jax
import jax.numpy as jnp
from jax import lax
from jax.experimental import pallas as pl
from jax.experimental.pallas import tpu as pltpu
from jax.experimental.pallas import tpu_sc as plsc

TIME_MAX = 10000
EMBED = 64
LANES = 16

CHUNK = 128


def _make_kernel(n_rows: int, seq_len: int, n_workers: int):
    rows_per_w = n_rows // n_workers
    n_chunks = rows_per_w // CHUNK
    mesh = plsc.VectorSubcoreMesh(core_axis_name="c", subcore_axis_name="s")

    @functools.partial(
        pl.kernel,
        out_type=jax.ShapeDtypeStruct((n_rows, EMBED), jnp.float32),
        mesh=mesh,
        scratch_types=[
            pltpu.VMEM((CHUNK,), jnp.int32),
            pltpu.VMEM((CHUNK,), jnp.int32),
            pltpu.VMEM((CHUNK, EMBED), jnp.float32),
            pltpu.VMEM((CHUNK, EMBED), jnp.float32),
            pltpu.VMEM((seq_len, EMBED), jnp.float32),
            pltpu.SemaphoreType.DMA,
        ],
        compiler_params=pltpu.CompilerParams(use_tc_tiling_on_sc=False),
    )
    def kern(time_hbm, event_hbm, tt_hbm, et_hbm, pt_hbm, out_hbm,
             idx_t, idx_e, buf_t, buf_e, pos_v, sem):
        n_cores = lax.axis_size("c")
        wid = lax.axis_index("s") * n_cores + lax.axis_index("c")
        w_base = wid * rows_per_w

        pltpu.sync_copy(pt_hbm.at[pl.ds(0, seq_len)], pos_v)

        @pl.loop(0, n_chunks)
        def _chunk(k):
            base = pl.multiple_of(w_base + k * CHUNK, CHUNK)
            pltpu.sync_copy(time_hbm.at[pl.ds(base, CHUNK)], idx_t)
            pltpu.sync_copy(event_hbm.at[pl.ds(base, CHUNK)], idx_e)

            for j in range(CHUNK // LANES):
                sl = pl.ds(j * LANES, LANES)
                idx_t[sl] = jnp.minimum(idx_t[sl], TIME_MAX)

            ct = pltpu.async_copy(tt_hbm.at[idx_t], buf_t, sem)
            ce = pltpu.async_copy(et_hbm.at[idx_e], buf_e, sem)
            ct.wait()
            ce.wait()

            @pl.loop(0, CHUNK)
            def _row(r):
                pr = lax.rem(base + r, seq_len)
                for c in range(EMBED // LANES):
                    sl = pl.ds(c * LANES, LANES)
                    buf_t[r, sl] = buf_t[r, sl] + buf_e[r, sl] + pos_v[pr, sl]

            pltpu.sync_copy(buf_t, out_hbm.at[pl.ds(base, CHUNK)])

    return kern


@jax.jit
def kernel(time, event, time_table, event_table, pos_table):
    b, l = event.shape
    n_rows = b * l
    info = plsc.get_sparse_core_info()
    n_workers = info.num_cores * info.num_subcores

    time_flat = time.astype(jnp.int32).reshape(n_rows)
    event_flat = event.astype(jnp.int32).reshape(n_rows)

    kern = _make_kernel(n_rows, l, n_workers)
    out = kern(time_flat, event_flat, time_table, event_table, pos_table)
    return out.reshape(b, l, EMBED)


# speedup vs baseline: 5.5292x; 1.4023x over previous
import functools

import jax
import jax.numpy as jnp
from jax import lax
from jax.experimental import pallas as pl
from jax.experimental.pallas import tpu as pltpu
from jax.experimental.pallas import tpu_sc as plsc

TIME_MAX = 10000
EMBED = 64
LANES = 16

CHUNK = 128


def _make_kernel(n_rows: int, seq_len: int, n_workers: int):
    rows_per_w = n_rows // n_workers
    n_chunks = rows_per_w // CHUNK
    mesh = plsc.VectorSubcoreMesh(core_axis_name="c", subcore_axis_name="s")

    @functools.partial(
        pl.kernel,
        out_type=jax.ShapeDtypeStruct((n_rows, EMBED), jnp.float32),
        mesh=mesh,
        scratch_types=[
            pltpu.VMEM((n_chunks, CHUNK), jnp.int32),
            pltpu.VMEM((n_chunks, CHUNK), jnp.int32),
            pltpu.VMEM((2, CHUNK, EMBED), jnp.float32),
            pltpu.VMEM((2, CHUNK, EMBED), jnp.float32),
            pltpu.VMEM((seq_len, EMBED), jnp.float32),
            pltpu.SemaphoreType.DMA,
            pltpu.SemaphoreType.DMA,
            pltpu.SemaphoreType.DMA,
            pltpu.SemaphoreType.DMA,
        ],
        compiler_params=pltpu.CompilerParams(use_tc_tiling_on_sc=False),
    )
    def kern(time_hbm, event_hbm, tt_hbm, et_hbm, pt_hbm, out_hbm,
             idx_t, idx_e, buf_t, buf_e, pos_v,
             sem_g0, sem_g1, sem_o0, sem_o1):
        n_cores = lax.axis_size("c")
        wid = lax.axis_index("s") * n_cores + lax.axis_index("c")
        w_base = pl.multiple_of(wid * rows_per_w, CHUNK)
        w_chunk = wid * n_chunks

        sem_g = (sem_g0, sem_g1)
        sem_o = (sem_o0, sem_o1)

        pltpu.sync_copy(pt_hbm.at[pl.ds(0, seq_len)], pos_v)
        pltpu.sync_copy(time_hbm.at[pl.ds(w_chunk, n_chunks)], idx_t)
        pltpu.sync_copy(event_hbm.at[pl.ds(w_chunk, n_chunks)], idx_e)

        def clamp(kk):
            for j in range(CHUNK // LANES):
                sl = pl.ds(j * LANES, LANES)
                idx_t[kk, sl] = jnp.minimum(idx_t[kk, sl], TIME_MAX)

        def start_gathers(kk, b):
            pltpu.async_copy(tt_hbm.at[idx_t.at[kk]], buf_t.at[b], sem_g[b])
            pltpu.async_copy(et_hbm.at[idx_e.at[kk]], buf_e.at[b], sem_g[b])

        def wait_gathers(b):
            pltpu.make_async_copy(tt_hbm.at[idx_t.at[0]],
                                  buf_t.at[b], sem_g[b]).wait()
            pltpu.make_async_copy(et_hbm.at[idx_e.at[0]],
                                  buf_e.at[b], sem_g[b]).wait()

        def wait_out(b):
            pltpu.make_async_copy(buf_t.at[b],
                                  out_hbm.at[pl.ds(0, CHUNK)],
                                  sem_o[b]).wait()

        clamp(0)
        start_gathers(0, 0)

        @pl.loop(0, n_chunks, step=2)
        def _pair(k):
            for b in (0, 1):
                kk = k + b
                ob = 1 - b
                nxt = kk + 1

                @pl.when(nxt < n_chunks)
                def _():
                    clamp(nxt)

                    @pl.when(nxt >= 2)
                    def _():
                        wait_out(ob)

                    start_gathers(nxt, ob)

                wait_gathers(b)
                base = pl.multiple_of(kk * CHUNK, CHUNK)

                @pl.loop(0, CHUNK)
                def _row(r):
                    pr = lax.rem(w_base + base + r, seq_len)
                    for c in range(EMBED // LANES):
                        sl = pl.ds(c * LANES, LANES)
                        buf_t[b, r, sl] = (buf_t[b, r, sl] + buf_e[b, r, sl]
                                           + pos_v[pr, sl])

                pltpu.async_copy(buf_t.at[b],
                                 out_hbm.at[pl.ds(w_base + base, CHUNK)],
                                 sem_o[b])

        wait_out(0)
        wait_out(1)

    return kern


@jax.jit
def kernel(time, event, time_table, event_table, pos_table):
    b, l = event.shape
    n_rows = b * l
    info = plsc.get_sparse_core_info()
    n_workers = info.num_cores * info.num_subcores

    time_2d = time.astype(jnp.int32).reshape(n_rows // CHUNK, CHUNK)
    event_2d = event.astype(jnp.int32).reshape(n_rows // CHUNK, CHUNK)

    kern = _make_kernel(n_rows, l, n_workers)
    out = kern(time_2d, event_2d, time_table, event_table, pos_table)
    return out.reshape(b, l, EMBED)


# speedup vs baseline: 18.4232x; 3.3320x over previous
import functools

import jax
import jax.numpy as jnp
from jax import lax
from jax.experimental import pallas as pl
from jax.experimental.pallas import tpu as pltpu
from jax.experimental.pallas import tpu_sc as plsc

TIME_MAX = 10000
EMBED = 64
LANES = 16
BLK = 128


def _make_kernel(n_batch: int, seq_len: int, n_workers: int):
    n_blk = n_batch // BLK
    assert n_blk == n_workers
    mesh = plsc.VectorSubcoreMesh(core_axis_name="c", subcore_axis_name="s")

    @functools.partial(
        pl.kernel,
        out_type=jax.ShapeDtypeStruct(
            (seq_len, EMBED // 8, n_blk, 8, BLK), jnp.float32),
        mesh=mesh,
        scratch_types=[
            pltpu.VMEM((seq_len, BLK), jnp.int32),
            pltpu.VMEM((seq_len, BLK), jnp.int32),
            pltpu.VMEM((2, BLK, EMBED), jnp.float32),
            pltpu.VMEM((2, BLK, EMBED), jnp.float32),
            pltpu.VMEM((2, EMBED // 8, 8, BLK + 1), jnp.float32),
            pltpu.VMEM((seq_len, EMBED), jnp.float32),
            pltpu.SemaphoreType.DMA,
            pltpu.SemaphoreType.DMA,
            pltpu.SemaphoreType.DMA,
            pltpu.SemaphoreType.DMA,
        ],
        compiler_params=pltpu.CompilerParams(use_tc_tiling_on_sc=False,
                                             needs_layout_passes=False),
    )
    def kern(time_hbm, event_hbm, tt_hbm, et_hbm, pt_hbm, out_hbm,
             idx_t, idx_e, buf_t, buf_e, obuf, pos_v,
             sem_g0, sem_g1, sem_o0, sem_o1):
        n_cores = lax.axis_size("c")
        wid = lax.axis_index("s") * n_cores + lax.axis_index("c")
        col = pl.multiple_of(wid * BLK, BLK)

        sem_g = (sem_g0, sem_g1)
        sem_o = (sem_o0, sem_o1)

        pltpu.sync_copy(pt_hbm.at[pl.ds(0, seq_len)], pos_v)
        pltpu.sync_copy(time_hbm.at[:, pl.ds(col, BLK)], idx_t)
        pltpu.sync_copy(event_hbm.at[:, pl.ds(col, BLK)], idx_e)

        def clamp(ll):
            for j in range(BLK // LANES):
                sl = pl.ds(j * LANES, LANES)
                idx_t[ll, sl] = jnp.minimum(idx_t[ll, sl], TIME_MAX)

        HALF = BLK // 2

        def start_gathers(ll, b):
            for hh in (0, HALF):
                pltpu.async_copy(tt_hbm.at[idx_t.at[ll, pl.ds(hh, HALF)]],
                                 buf_t.at[b, pl.ds(hh, HALF)], sem_g[b])
                pltpu.async_copy(et_hbm.at[idx_e.at[ll, pl.ds(hh, HALF)]],
                                 buf_e.at[b, pl.ds(hh, HALF)], sem_g[b])

        def wait_gathers(b):
            for hh in (0, HALF):
                pltpu.make_async_copy(tt_hbm.at[idx_t.at[0, pl.ds(hh, HALF)]],
                                      buf_t.at[b, pl.ds(hh, HALF)],
                                      sem_g[b]).wait()
                pltpu.make_async_copy(et_hbm.at[idx_e.at[0, pl.ds(hh, HALF)]],
                                      buf_e.at[b, pl.ds(hh, HALF)],
                                      sem_g[b]).wait()

        def wait_out(b):
            pltpu.make_async_copy(obuf.at[b, :, :, pl.ds(0, BLK)],
                                  out_hbm.at[0, :, 0], sem_o[b]).wait()

        lane = lax.iota(jnp.int32, LANES)
        eb_vecs = [(lane + q * LANES) >> 3 for q in range(EMBED // LANES)]
        ei_vecs = [(lane + q * LANES) & 7 for q in range(EMBED // LANES)]

        clamp(0)
        start_gathers(0, 0)

        @pl.loop(0, seq_len, step=2)
        def _pair(k):
            for b in (0, 1):
                ll = k + b
                ob = 1 - b
                nxt = ll + 1

                @pl.when(nxt < seq_len)
                def _():
                    clamp(nxt)
                    start_gathers(nxt, ob)

                wait_gathers(b)

                @pl.when(ll >= 2)
                def _():
                    wait_out(b)

                pvs = [pos_v[ll, pl.ds(q * LANES, LANES)]
                       for q in range(EMBED // LANES)]

                @plsc.parallel_loop(0, BLK, unroll=4)
                def _row(r):
                    cb = jnp.broadcast_to(r, (LANES,)).astype(jnp.int32)
                    for q in range(EMBED // LANES):
                        sl = pl.ds(q * LANES, LANES)
                        v = buf_t[b, r, sl] + buf_e[b, r, sl] + pvs[q]
                        plsc.store_scatter(obuf.at[b],
                                           [eb_vecs[q], ei_vecs[q], cb], v)

                pltpu.async_copy(obuf.at[b, :, :, pl.ds(0, BLK)],
                                 out_hbm.at[ll, :, wid], sem_o[b])

        wait_out(0)
        wait_out(1)

    return kern


@jax.jit
def kernel(time, event, time_table, event_table, pos_table):
    b, l = event.shape
    info = plsc.get_sparse_core_info()
    n_workers = info.num_cores * info.num_subcores

    kern = _make_kernel(b, l, n_workers)
    x5 = kern(time.astype(jnp.int32).T, event.astype(jnp.int32).T,
              time_table, event_table, pos_table)
    return x5.transpose(2, 4, 0, 1, 3).reshape(b, l, EMBED)
